# baseline (speedup 1.0000x reference)
import functools
import jax
import jax.numpy as jnp
from jax.experimental import pallas as pl
from jax.experimental.pallas import tpu as pltpu

_NEG = -1e30


def _qkv_body(x_ref, nw_ref, qw_ref, kw_ref, vw_ref, q_ref, k_ref, v_ref):
    xb = x_ref[...]
    var = jnp.mean(xb * xb, axis=1, keepdims=True)
    h = (nw_ref[...] * xb) * jax.lax.rsqrt(var + 1e-6)
    q_ref[...] = jnp.dot(h, qw_ref[...], preferred_element_type=jnp.float32)
    k_ref[...] = jnp.dot(h, kw_ref[...], preferred_element_type=jnp.float32)
    v_ref[...] = jnp.dot(h, vw_ref[...], preferred_element_type=jnp.float32)


def _qkv(xf, attn_norm_w, q_w, k_w, v_w, tm):
    n, c = xf.shape
    grid = (n // tm,)
    wspec = pl.BlockSpec((c, c), lambda i: (0, 0))
    return pl.pallas_call(
        _qkv_body,
        grid=grid,
        in_specs=[
            pl.BlockSpec((tm, c), lambda i: (i, 0)),
            pl.BlockSpec((1, c), lambda i: (0, 0)),
            wspec, wspec, wspec,
        ],
        out_specs=[pl.BlockSpec((tm, c), lambda i: (i, 0))] * 3,
        out_shape=[jax.ShapeDtypeStruct((n, c), jnp.float32)] * 3,
        compiler_params=pltpu.CompilerParams(
            dimension_semantics=("parallel",)),
    )(xf, attn_norm_w.reshape(1, c), q_w, k_w, v_w)


def _qkvr_body(x_ref, nw_ref, qw_ref, kw_ref, vw_ref, c_ref, s_ref,
               qo_ref, ko_ref, vx_ref):
    xb = x_ref[0]
    var = jnp.mean(xb * xb, axis=1, keepdims=True)
    hh = (nw_ref[...] * xb) * jax.lax.rsqrt(var + 1e-6)
    q = jnp.dot(hh, qw_ref[...], preferred_element_type=jnp.float32)
    k = jnp.dot(hh, kw_ref[...], preferred_element_type=jnp.float32)
    v = jnp.dot(hh, vw_ref[...], preferred_element_type=jnp.float32)
    tm, c = q.shape
    hd = c_ref.shape[-1]
    nh = c // hd
    cos = c_ref[...][:, None, :]
    sin = s_ref[...][:, None, :]

    def rot(x2):
        x3 = x2.reshape(tm, nh, hd)
        x1 = x3[..., : hd // 2]
        xr = x3[..., hd // 2:]
        r = jnp.concatenate([-xr, x1], axis=-1)
        return (x3 * cos + r * sin).transpose(1, 0, 2)

    qo_ref[0] = rot(q)
    ko_ref[0] = rot(k)
    vt = v.reshape(tm, nh, hd).transpose(1, 0, 2)
    lane = jax.lax.broadcasted_iota(jnp.int32, (nh, tm, 2 * hd), 2)
    vx_ref[0] = jnp.where(lane < hd,
                          jnp.pad(vt, ((0, 0), (0, 0), (0, hd))),
                          jnp.where(lane == hd, 1.0, 0.0))


def _qkv_rope(x3, attn_norm_w, q_w, k_w, v_w, cos, sin, tm, nh):
    b, t, c = x3.shape
    dh = c // nh
    wspec = pl.BlockSpec((c, c), lambda bb, i: (0, 0))
    cspec = pl.BlockSpec((tm, dh), lambda bb, i: (i, 0))
    ospec = pl.BlockSpec((1, nh, tm, dh), lambda bb, i: (bb, 0, i, 0))
    vxspec = pl.BlockSpec((1, nh, tm, 2 * dh), lambda bb, i: (bb, 0, i, 0))
    return pl.pallas_call(
        _qkvr_body,
        grid=(b, t // tm),
        in_specs=[
            pl.BlockSpec((1, tm, c), lambda bb, i: (bb, i, 0)),
            pl.BlockSpec((1, c), lambda bb, i: (0, 0)),
            wspec, wspec, wspec, cspec, cspec,
        ],
        out_specs=[ospec, ospec, vxspec],
        out_shape=[jax.ShapeDtypeStruct((b, nh, t, dh), jnp.float32)] * 2
        + [jax.ShapeDtypeStruct((b, nh, t, 2 * dh), jnp.float32)],
        compiler_params=pltpu.CompilerParams(
            dimension_semantics=("parallel", "parallel")),
    )(x3, attn_norm_w.reshape(1, c), q_w, k_w, v_w, cos, sin)


def _flash_body(q_ref, k_ref, vx_ref, o_ref, acc_ref, *, scale, tq, tk):
    i = pl.program_id(1)
    j = pl.program_id(2)

    @pl.when(j == 0)
    def _init():
        acc_ref[...] = jnp.zeros_like(acc_ref)

    @pl.when(j * tk <= i * tq + tq - 1)
    def _compute():
        s = jax.lax.dot_general(q_ref[0], k_ref[0], (((1,), (1,)), ((), ())),
                                preferred_element_type=jnp.float32) * scale
        rows = jax.lax.broadcasted_iota(jnp.int32, (tq, tk), 0) + i * tq
        cols = jax.lax.broadcasted_iota(jnp.int32, (tq, tk), 1) + j * tk
        p = jnp.exp(jnp.where(rows >= cols, s, _NEG))
        acc_ref[...] += jnp.dot(p, vx_ref[0],
                                preferred_element_type=jnp.float32)

    @pl.when(j == pl.num_programs(2) - 1)
    def _finish():
        dh = o_ref.shape[-1]
        acc = acc_ref[...]
        o_ref[0] = acc[:, :dh] / acc[:, dh:dh + 1]


def _flash(q, k, vx, tq, tk):
    bh, t, dh = q.shape
    scale = 1.0 / (dh ** 0.5)
    grid = (bh, t // tq, t // tk)
    body = functools.partial(_flash_body, scale=scale, tq=tq, tk=tk)
    return pl.pallas_call(
        body,
        grid=grid,
        in_specs=[
            pl.BlockSpec((1, tq, dh), lambda w, i, j: (w, i, 0)),
            pl.BlockSpec((1, tk, dh), lambda w, i, j: (w, j, 0)),
            pl.BlockSpec((1, tk, 2 * dh), lambda w, i, j: (w, j, 0)),
        ],
        out_specs=pl.BlockSpec((1, tq, dh), lambda w, i, j: (w, i, 0)),
        out_shape=jax.ShapeDtypeStruct((bh, t, dh), jnp.float32),
        scratch_shapes=[
            pltpu.VMEM((tq, 2 * dh), jnp.float32),
        ],
        compiler_params=pltpu.CompilerParams(
            dimension_semantics=("parallel", "parallel", "arbitrary")),
    )(q, k, vx)


def _post_body(x_ref, y_ref, ow_ref, fw_ref, rw_ref,
               x2_ref, hm_ref, idx_ref, wt_ref, *, e):
    xb = x_ref[...]
    tm = xb.shape[0]
    yb = y_ref[0].transpose(1, 0, 2).reshape(tm, -1)
    x2 = xb + jnp.dot(yb, ow_ref[...],
                      preferred_element_type=jnp.float32)
    var = jnp.mean(x2 * x2, axis=1, keepdims=True)
    hm = (fw_ref[...] * x2) * jax.lax.rsqrt(var + 1e-6)
    logits = jnp.dot(hm, rw_ref[...], preferred_element_type=jnp.float32)
    tm, lanes = logits.shape
    colid = jax.lax.broadcasted_iota(jnp.int32, (tm, lanes), 1)
    lg = jnp.where(colid < e, logits, _NEG)
    m1 = jnp.max(lg, axis=1, keepdims=True)
    i1 = jnp.min(jnp.where(lg == m1, colid, 999), axis=1, keepdims=True)
    lg2 = jnp.where(colid == i1, _NEG, lg)
    m2 = jnp.max(lg2, axis=1, keepdims=True)
    i2 = jnp.min(jnp.where(lg2 == m2, colid, 999), axis=1, keepdims=True)
    e2 = jnp.exp(m2 - m1)
    w1v = 1.0 / (1.0 + e2)
    w2v = e2 / (1.0 + e2)
    x2_ref[...] = x2
    hm_ref[...] = hm
    idx_ref[...] = jnp.where(colid == 0, i1, jnp.where(colid == 1, i2, 0))
    wt_ref[...] = jnp.where(colid == 0, w1v, jnp.where(colid == 1, w2v, 0.0))


def _post(xf, yf, o_w, ffn_norm_w, router_w, tm):
    n, c = xf.shape
    b, nh, t, dh = yf.shape
    e = router_w.shape[1]
    rw = jnp.pad(router_w, ((0, 0), (0, 128 - e)))
    grid = (n // tm,)
    tpb = t // tm
    body = functools.partial(_post_body, e=e)
    return pl.pallas_call(
        body,
        grid=grid,
        in_specs=[
            pl.BlockSpec((tm, c), lambda i: (i, 0)),
            pl.BlockSpec((1, nh, tm, dh), lambda i: (i // tpb, 0, i % tpb, 0)),
            pl.BlockSpec((c, c), lambda i: (0, 0)),
            pl.BlockSpec((1, c), lambda i: (0, 0)),
            pl.BlockSpec((c, 128), lambda i: (0, 0)),
        ],
        out_specs=[
            pl.BlockSpec((tm, c), lambda i: (i, 0)),
            pl.BlockSpec((tm, c), lambda i: (i, 0)),
            pl.BlockSpec((tm, 128), lambda i: (i, 0)),
            pl.BlockSpec((tm, 128), lambda i: (i, 0)),
        ],
        out_shape=[
            jax.ShapeDtypeStruct((n, c), jnp.float32),
            jax.ShapeDtypeStruct((n, c), jnp.float32),
            jax.ShapeDtypeStruct((n, 128), jnp.int32),
            jax.ShapeDtypeStruct((n, 128), jnp.float32),
        ],
        compiler_params=pltpu.CompilerParams(
            dimension_semantics=("parallel",)),
    )(xf, yf, o_w, ffn_norm_w.reshape(1, c), rw)


def _meta_body(i0_ref, i1_ref, d0_ref, d1_ref, tx_ref, *, ne, tm):
    i0 = i0_ref[...]
    i1 = i1_ref[...]
    r, lanes = i0.shape
    ci = jax.lax.broadcasted_iota(jnp.int32, (lanes, lanes), 0)
    cj = jax.lax.broadcasted_iota(jnp.int32, (lanes, lanes), 1)
    ustrict = jnp.where(ci < cj, 1.0, 0.0)
    ri = jax.lax.broadcasted_iota(jnp.int32, (r, r), 0)
    rj = jax.lax.broadcasted_iota(jnp.int32, (r, r), 1)
    lstrict = jnp.where(rj < ri, 1.0, 0.0)
    d0 = jnp.zeros((r, lanes), jnp.float32)
    d1 = jnp.zeros((r, lanes), jnp.float32)
    tilei = (jax.lax.broadcasted_iota(jnp.int32, (1, 128), 1)
             .astype(jnp.float32) * tm)
    tx = jnp.zeros((1, 128), jnp.float32)
    poff = 0.0
    for e in range(ne):
        m0 = jnp.where(i0 == e, 1.0, 0.0)
        m1 = jnp.where(i1 == e, 1.0, 0.0)
        mm = m0 + m1
        colex = jnp.dot(mm, ustrict, preferred_element_type=jnp.float32)
        rowsum = jnp.sum(mm, axis=1, keepdims=True)
        rowex = jnp.dot(lstrict, rowsum, preferred_element_type=jnp.float32)
        cnt_before = colex + rowex
        count = jnp.sum(mm)
        d0 = jnp.where(i0 == e, poff + cnt_before, d0)
        d1 = jnp.where(i1 == e, poff + cnt_before + m0, d1)
        poff = poff + jnp.ceil(count / tm) * tm
        tx = tx + jnp.where(tilei >= poff, 1.0, 0.0)
    d0_ref[...] = d0.astype(jnp.int32)
    d1_ref[...] = d1.astype(jnp.int32)
    tx_ref[...] = jnp.minimum(tx, ne - 1).astype(jnp.int32)


def _dispatch_meta(i0, i1, ne, tm):
    r, lanes = i0.shape
    body = functools.partial(_meta_body, ne=ne, tm=tm)
    return pl.pallas_call(
        body,
        out_shape=[
            jax.ShapeDtypeStruct((r, lanes), jnp.int32),
            jax.ShapeDtypeStruct((r, lanes), jnp.int32),
            jax.ShapeDtypeStruct((1, 128), jnp.int32),
        ],
    )(i0, i1)


def _gffn_body(texp_ref, hm_ref, w1_ref, w3_ref, w2_ref, out_ref):
    hmb = hm_ref[...]
    a = jnp.dot(hmb, w1_ref[0], preferred_element_type=jnp.float32)
    b3 = jnp.dot(hmb, w3_ref[0], preferred_element_type=jnp.float32)
    act = (a * (1.0 / (1.0 + jnp.exp(-a)))) * b3
    out_ref[...] = jnp.dot(act, w2_ref[0], preferred_element_type=jnp.float32)


def _grouped_ffn(hm_g, texp, w1, w2, w3, tm):
    pad_n, c = hm_g.shape
    ne, _, hid = w1.shape
    mt = pad_n // tm
    grid_spec = pltpu.PrefetchScalarGridSpec(
        num_scalar_prefetch=1,
        grid=(mt,),
        in_specs=[
            pl.BlockSpec((tm, c), lambda i, texp: (i, 0)),
            pl.BlockSpec((1, c, hid), lambda i, texp: (texp[i], 0, 0)),
            pl.BlockSpec((1, c, hid), lambda i, texp: (texp[i], 0, 0)),
            pl.BlockSpec((1, hid, c), lambda i, texp: (texp[i], 0, 0)),
        ],
        out_specs=pl.BlockSpec((tm, c), lambda i, texp: (i, 0)),
    )
    return pl.pallas_call(
        _gffn_body,
        grid_spec=grid_spec,
        out_shape=jax.ShapeDtypeStruct((pad_n, c), jnp.float32),
        compiler_params=pltpu.CompilerParams(
            dimension_semantics=("arbitrary",)),
    )(texp, hm_g, w1, w3, w2)


def _combine_body(x2_ref, g0_ref, g1_ref, wt_ref, out_ref):
    w0 = wt_ref[...][:, :1]
    w1 = wt_ref[...][:, 1:2]
    out_ref[...] = x2_ref[...] + w0 * g0_ref[...] + w1 * g1_ref[...]


def _combine(x2, g0, g1, wt2, tm):
    n, c = x2.shape
    spec = pl.BlockSpec((tm, c), lambda i: (i, 0))
    return pl.pallas_call(
        _combine_body,
        grid=(n // tm,),
        in_specs=[spec, spec, spec,
                  pl.BlockSpec((tm, 128), lambda i: (i, 0))],
        out_specs=spec,
        out_shape=jax.ShapeDtypeStruct((n, c), jnp.float32),
        compiler_params=pltpu.CompilerParams(
            dimension_semantics=("parallel",)),
    )(x2, g0, g1, wt2)


from jax.experimental.pallas import tpu_sc as plsc


def _sc_dispatch(hm, d0, d1, pad_n):
    n, c = hm.shape
    nw, chunks, ch = d0.shape
    info = plsc.get_sparse_core_info()
    nc = info.num_cores
    mesh = plsc.VectorSubcoreMesh(core_axis_name="c", subcore_axis_name="s")

    @functools.partial(
        pl.kernel, mesh=mesh,
        out_type=jax.ShapeDtypeStruct((pad_n, c), jnp.float32),
        scratch_types=[
            pltpu.VMEM((chunks, ch), jnp.int32),
            pltpu.VMEM((chunks, ch), jnp.int32),
            pltpu.VMEM((ch, c), jnp.float32),
            pltpu.SemaphoreType.DMA,
        ],
    )
    def k(hm_hbm, d0_hbm, d1_hbm, out_hbm, i0_v, i1_v, rows_v, sem):
        wid = jax.lax.axis_index("s") * nc + jax.lax.axis_index("c")
        pltpu.sync_copy(d0_hbm.at[wid], i0_v)
        pltpu.sync_copy(d1_hbm.at[wid], i1_v)
        rows_per_w = chunks * ch

        def body(ci, carry):
            base = wid * rows_per_w + ci * ch
            pltpu.sync_copy(hm_hbm.at[pl.ds(base, ch)], rows_v)
            pltpu.async_copy(rows_v, out_hbm.at[i0_v.at[ci]], sem).wait()
            pltpu.async_copy(rows_v, out_hbm.at[i1_v.at[ci]], sem).wait()
            return carry

        jax.lax.fori_loop(0, chunks, body, 0)

    return k(hm, d0, d1)


def _sc_gather(table, idx3):
    nw, chunks, ch = idx3.shape
    v, d = table.shape
    bsz = nw * chunks * ch
    info = plsc.get_sparse_core_info()
    nc = info.num_cores
    mesh = plsc.VectorSubcoreMesh(core_axis_name="c", subcore_axis_name="s")

    @functools.partial(
        pl.kernel, mesh=mesh,
        out_type=jax.ShapeDtypeStruct((bsz, d), jnp.float32),
        scratch_types=[
            pltpu.VMEM((chunks, ch), jnp.int32),
            pltpu.VMEM((ch, d), jnp.float32),
            pltpu.SemaphoreType.DMA,
        ],
    )
    def k(table_hbm, idx_hbm, out_hbm, idx_v, rows_v, sem):
        wid = jax.lax.axis_index("s") * nc + jax.lax.axis_index("c")
        pltpu.sync_copy(idx_hbm.at[wid], idx_v)
        base = wid * (chunks * ch)

        def body(ci, carry):
            pltpu.async_copy(table_hbm.at[idx_v.at[ci]], rows_v, sem).wait()
            pltpu.sync_copy(rows_v, out_hbm.at[pl.ds(base + ci * ch, ch)])
            return carry

        jax.lax.fori_loop(0, chunks, body, 0)

    return k(table, idx3)


def kernel(x, rope_cos, rope_sin, attn_norm_w, q_w, k_w, v_w, o_w,
           ffn_norm_w, router_w, w1, w2, w3):
    b, t, c = x.shape
    dh = rope_cos.shape[1]
    h = c // dh
    n = b * t
    ne = router_w.shape[1]

    xf = x.reshape(n, c)
    tm = min(512, t)
    qh, kh, vx = _qkv_rope(x, attn_norm_w, q_w, k_w, v_w,
                           rope_cos, rope_sin, tm, h)
    tq = min(1024, t)
    tk = min(1024, t)
    y = _flash(qh.reshape(b * h, t, dh), kh.reshape(b * h, t, dh),
               vx.reshape(b * h, t, 2 * dh), tq, tk)
    yf = y

    x2, hm, idx2, wt2 = _post(xf, yf.reshape(b, h, t, dh), o_w, ffn_norm_w,
                              router_w, tm)

    tg = 256
    pad_n = 2 * n + ne * tg
    nw = 32
    i0 = idx2[:, 0].reshape(n // 128, 128)
    i1 = idx2[:, 1].reshape(n // 128, 128)
    d0, d1, tx = _dispatch_meta(i0, i1, ne, tg)
    texp = tx[0, : pad_n // tg]
    rpw = n // nw
    ch = min(32, rpw)
    hm_g = _sc_dispatch(hm, d0.reshape(nw, rpw // ch, ch),
                        d1.reshape(nw, rpw // ch, ch), pad_n)
    rows = _grouped_ffn(hm_g, texp, w1, w2, w3, tg)
    p01 = jnp.concatenate([d0.reshape(n), d1.reshape(n)])
    ch2 = min(64, (2 * n) // nw)
    g = _sc_gather(rows, p01.reshape(nw, (2 * n) // (nw * ch2), ch2))
    out = _combine(x2, g[:n], g[n:], wt2, tm)
    return out.reshape(b, t, c)


# speedup vs baseline: 1.0186x; 1.0186x over previous
import functools
import jax
import jax.numpy as jnp
from jax.experimental import pallas as pl
from jax.experimental.pallas import tpu as pltpu

_NEG = -1e30


def _qkv_body(x_ref, nw_ref, qw_ref, kw_ref, vw_ref, q_ref, k_ref, v_ref):
    xb = x_ref[...]
    var = jnp.mean(xb * xb, axis=1, keepdims=True)
    h = (nw_ref[...] * xb) * jax.lax.rsqrt(var + 1e-6)
    q_ref[...] = jnp.dot(h, qw_ref[...], preferred_element_type=jnp.float32)
    k_ref[...] = jnp.dot(h, kw_ref[...], preferred_element_type=jnp.float32)
    v_ref[...] = jnp.dot(h, vw_ref[...], preferred_element_type=jnp.float32)


def _qkv(xf, attn_norm_w, q_w, k_w, v_w, tm):
    n, c = xf.shape
    grid = (n // tm,)
    wspec = pl.BlockSpec((c, c), lambda i: (0, 0))
    return pl.pallas_call(
        _qkv_body,
        grid=grid,
        in_specs=[
            pl.BlockSpec((tm, c), lambda i: (i, 0)),
            pl.BlockSpec((1, c), lambda i: (0, 0)),
            wspec, wspec, wspec,
        ],
        out_specs=[pl.BlockSpec((tm, c), lambda i: (i, 0))] * 3,
        out_shape=[jax.ShapeDtypeStruct((n, c), jnp.float32)] * 3,
        compiler_params=pltpu.CompilerParams(
            dimension_semantics=("parallel",)),
    )(xf, attn_norm_w.reshape(1, c), q_w, k_w, v_w)


def _qkvr_body(x_ref, nw_ref, qw_ref, kw_ref, vw_ref, c_ref, s_ref,
               qo_ref, ko_ref, vx_ref):
    xb = x_ref[0]
    var = jnp.mean(xb * xb, axis=1, keepdims=True)
    hh = (nw_ref[...] * xb) * jax.lax.rsqrt(var + 1e-6)
    q = jnp.dot(hh, qw_ref[...], preferred_element_type=jnp.float32)
    k = jnp.dot(hh, kw_ref[...], preferred_element_type=jnp.float32)
    v = jnp.dot(hh, vw_ref[...], preferred_element_type=jnp.float32)
    tm, c = q.shape
    hd = c_ref.shape[-1]
    nh = c // hd
    cos = c_ref[...][:, None, :]
    sin = s_ref[...][:, None, :]

    def rot(x2):
        x3 = x2.reshape(tm, nh, hd)
        x1 = x3[..., : hd // 2]
        xr = x3[..., hd // 2:]
        r = jnp.concatenate([-xr, x1], axis=-1)
        return (x3 * cos + r * sin).transpose(1, 0, 2)

    qo_ref[0] = rot(q)
    ko_ref[0] = rot(k)
    vt = v.reshape(tm, nh, hd).transpose(1, 0, 2)
    lane = jax.lax.broadcasted_iota(jnp.int32, (nh, tm, 2 * hd), 2)
    vx_ref[0] = jnp.where(lane < hd,
                          jnp.pad(vt, ((0, 0), (0, 0), (0, hd))),
                          jnp.where(lane == hd, 1.0, 0.0))


def _qkv_rope(x3, attn_norm_w, q_w, k_w, v_w, cos, sin, tm, nh):
    b, t, c = x3.shape
    dh = c // nh
    wspec = pl.BlockSpec((c, c), lambda bb, i: (0, 0))
    cspec = pl.BlockSpec((tm, dh), lambda bb, i: (i, 0))
    ospec = pl.BlockSpec((1, nh, tm, dh), lambda bb, i: (bb, 0, i, 0))
    vxspec = pl.BlockSpec((1, nh, tm, 2 * dh), lambda bb, i: (bb, 0, i, 0))
    return pl.pallas_call(
        _qkvr_body,
        grid=(b, t // tm),
        in_specs=[
            pl.BlockSpec((1, tm, c), lambda bb, i: (bb, i, 0)),
            pl.BlockSpec((1, c), lambda bb, i: (0, 0)),
            wspec, wspec, wspec, cspec, cspec,
        ],
        out_specs=[ospec, ospec, vxspec],
        out_shape=[jax.ShapeDtypeStruct((b, nh, t, dh), jnp.float32)] * 2
        + [jax.ShapeDtypeStruct((b, nh, t, 2 * dh), jnp.float32)],
        compiler_params=pltpu.CompilerParams(
            dimension_semantics=("parallel", "parallel")),
    )(x3, attn_norm_w.reshape(1, c), q_w, k_w, v_w, cos, sin)


def _flash_body(q_ref, k_ref, vx_ref, o_ref, acc_ref, *, scale, tq, tk):
    i = pl.program_id(1)
    j = pl.program_id(2)

    @pl.when(j == 0)
    def _init():
        acc_ref[...] = jnp.zeros_like(acc_ref)

    @pl.when(j * tk <= i * tq + tq - 1)
    def _compute():
        s = jax.lax.dot_general(q_ref[0], k_ref[0], (((1,), (1,)), ((), ())),
                                preferred_element_type=jnp.float32) * scale
        rows = jax.lax.broadcasted_iota(jnp.int32, (tq, tk), 0) + i * tq
        cols = jax.lax.broadcasted_iota(jnp.int32, (tq, tk), 1) + j * tk
        p = jnp.exp(jnp.where(rows >= cols, s, _NEG))
        acc_ref[...] += jnp.dot(p, vx_ref[0],
                                preferred_element_type=jnp.float32)

    @pl.when(j == pl.num_programs(2) - 1)
    def _finish():
        dh = o_ref.shape[-1]
        acc = acc_ref[...]
        o_ref[0] = acc[:, :dh] / acc[:, dh:dh + 1]


def _flash(q, k, vx, tq, tk):
    bh, t, dh = q.shape
    scale = 1.0 / (dh ** 0.5)
    grid = (bh, t // tq, t // tk)
    body = functools.partial(_flash_body, scale=scale, tq=tq, tk=tk)
    return pl.pallas_call(
        body,
        grid=grid,
        in_specs=[
            pl.BlockSpec((1, tq, dh), lambda w, i, j: (w, i, 0)),
            pl.BlockSpec((1, tk, dh), lambda w, i, j: (w, j, 0)),
            pl.BlockSpec((1, tk, 2 * dh), lambda w, i, j: (w, j, 0)),
        ],
        out_specs=pl.BlockSpec((1, tq, dh), lambda w, i, j: (w, i, 0)),
        out_shape=jax.ShapeDtypeStruct((bh, t, dh), jnp.float32),
        scratch_shapes=[
            pltpu.VMEM((tq, 2 * dh), jnp.float32),
        ],
        compiler_params=pltpu.CompilerParams(
            dimension_semantics=("parallel", "parallel", "arbitrary")),
    )(q, k, vx)


def _post_body(x_ref, y_ref, ow_ref, fw_ref, rw_ref,
               x2_ref, hm_ref, idx_ref, wt_ref, *, e):
    xb = x_ref[...]
    tm = xb.shape[0]
    yb = y_ref[0].transpose(1, 0, 2).reshape(tm, -1)
    x2 = xb + jnp.dot(yb, ow_ref[...],
                      preferred_element_type=jnp.float32)
    var = jnp.mean(x2 * x2, axis=1, keepdims=True)
    hm = (fw_ref[...] * x2) * jax.lax.rsqrt(var + 1e-6)
    logits = jnp.dot(hm, rw_ref[...], preferred_element_type=jnp.float32)
    tm, lanes = logits.shape
    colid = jax.lax.broadcasted_iota(jnp.int32, (tm, lanes), 1)
    lg = jnp.where(colid < e, logits, _NEG)
    m1 = jnp.max(lg, axis=1, keepdims=True)
    i1 = jnp.min(jnp.where(lg == m1, colid, 999), axis=1, keepdims=True)
    lg2 = jnp.where(colid == i1, _NEG, lg)
    m2 = jnp.max(lg2, axis=1, keepdims=True)
    i2 = jnp.min(jnp.where(lg2 == m2, colid, 999), axis=1, keepdims=True)
    e2 = jnp.exp(m2 - m1)
    w1v = 1.0 / (1.0 + e2)
    w2v = e2 / (1.0 + e2)
    x2_ref[...] = x2
    hm_ref[...] = hm
    idx_ref[...] = jnp.where(colid == 0, i1, jnp.where(colid == 1, i2, 0))
    wt_ref[...] = jnp.where(colid == 0, w1v, jnp.where(colid == 1, w2v, 0.0))


def _post(xf, yf, o_w, ffn_norm_w, router_w, tm):
    n, c = xf.shape
    b, nh, t, dh = yf.shape
    e = router_w.shape[1]
    rw = jnp.pad(router_w, ((0, 0), (0, 128 - e)))
    grid = (n // tm,)
    tpb = t // tm
    body = functools.partial(_post_body, e=e)
    return pl.pallas_call(
        body,
        grid=grid,
        in_specs=[
            pl.BlockSpec((tm, c), lambda i: (i, 0)),
            pl.BlockSpec((1, nh, tm, dh), lambda i: (i // tpb, 0, i % tpb, 0)),
            pl.BlockSpec((c, c), lambda i: (0, 0)),
            pl.BlockSpec((1, c), lambda i: (0, 0)),
            pl.BlockSpec((c, 128), lambda i: (0, 0)),
        ],
        out_specs=[
            pl.BlockSpec((tm, c), lambda i: (i, 0)),
            pl.BlockSpec((tm, c), lambda i: (i, 0)),
            pl.BlockSpec((tm, 128), lambda i: (i, 0)),
            pl.BlockSpec((tm, 128), lambda i: (i, 0)),
        ],
        out_shape=[
            jax.ShapeDtypeStruct((n, c), jnp.float32),
            jax.ShapeDtypeStruct((n, c), jnp.float32),
            jax.ShapeDtypeStruct((n, 128), jnp.int32),
            jax.ShapeDtypeStruct((n, 128), jnp.float32),
        ],
        compiler_params=pltpu.CompilerParams(
            dimension_semantics=("parallel",)),
    )(xf, yf, o_w, ffn_norm_w.reshape(1, c), rw)


def _meta_body(i0_ref, i1_ref, d0_ref, d1_ref, tx_ref, *, ne, tm):
    i0 = i0_ref[...]
    i1 = i1_ref[...]
    r, lanes = i0.shape
    ci = jax.lax.broadcasted_iota(jnp.int32, (lanes, lanes), 0)
    cj = jax.lax.broadcasted_iota(jnp.int32, (lanes, lanes), 1)
    ustrict = jnp.where(ci < cj, 1.0, 0.0)
    ri = jax.lax.broadcasted_iota(jnp.int32, (r, r), 0)
    rj = jax.lax.broadcasted_iota(jnp.int32, (r, r), 1)
    lstrict = jnp.where(rj < ri, 1.0, 0.0)
    d0 = jnp.zeros((r, lanes), jnp.float32)
    d1 = jnp.zeros((r, lanes), jnp.float32)
    tilei = (jax.lax.broadcasted_iota(jnp.int32, (1, 128), 1)
             .astype(jnp.float32) * tm)
    tx = jnp.zeros((1, 128), jnp.float32)
    poff = 0.0
    for e in range(ne):
        m0 = jnp.where(i0 == e, 1.0, 0.0)
        m1 = jnp.where(i1 == e, 1.0, 0.0)
        mm = m0 + m1
        colex = jnp.dot(mm, ustrict, preferred_element_type=jnp.float32)
        rowsum = jnp.sum(mm, axis=1, keepdims=True)
        rowex = jnp.dot(lstrict, rowsum, preferred_element_type=jnp.float32)
        cnt_before = colex + rowex
        count = jnp.sum(mm)
        d0 = jnp.where(i0 == e, poff + cnt_before, d0)
        d1 = jnp.where(i1 == e, poff + cnt_before + m0, d1)
        poff = poff + jnp.ceil(count / tm) * tm
        tx = tx + jnp.where(tilei >= poff, 1.0, 0.0)
    d0_ref[...] = d0.astype(jnp.int32)
    d1_ref[...] = d1.astype(jnp.int32)
    tx_ref[...] = jnp.minimum(tx, ne - 1).astype(jnp.int32)


def _dispatch_meta(i0, i1, ne, tm):
    r, lanes = i0.shape
    body = functools.partial(_meta_body, ne=ne, tm=tm)
    return pl.pallas_call(
        body,
        out_shape=[
            jax.ShapeDtypeStruct((r, lanes), jnp.int32),
            jax.ShapeDtypeStruct((r, lanes), jnp.int32),
            jax.ShapeDtypeStruct((1, 128), jnp.int32),
        ],
    )(i0, i1)


def _gffn_body(texp_ref, hm_ref, w1_ref, w3_ref, w2_ref, out_ref):
    hmb = hm_ref[...]
    a = jnp.dot(hmb, w1_ref[0], preferred_element_type=jnp.float32)
    b3 = jnp.dot(hmb, w3_ref[0], preferred_element_type=jnp.float32)
    act = (a * (1.0 / (1.0 + jnp.exp(-a)))) * b3
    out_ref[...] = jnp.dot(act, w2_ref[0], preferred_element_type=jnp.float32)


def _grouped_ffn(hm_g, texp, w1, w2, w3, tm):
    pad_n, c = hm_g.shape
    ne, _, hid = w1.shape
    mt = pad_n // tm
    grid_spec = pltpu.PrefetchScalarGridSpec(
        num_scalar_prefetch=1,
        grid=(mt,),
        in_specs=[
            pl.BlockSpec((tm, c), lambda i, texp: (i, 0)),
            pl.BlockSpec((1, c, hid), lambda i, texp: (texp[i], 0, 0)),
            pl.BlockSpec((1, c, hid), lambda i, texp: (texp[i], 0, 0)),
            pl.BlockSpec((1, hid, c), lambda i, texp: (texp[i], 0, 0)),
        ],
        out_specs=pl.BlockSpec((tm, c), lambda i, texp: (i, 0)),
    )
    return pl.pallas_call(
        _gffn_body,
        grid_spec=grid_spec,
        out_shape=jax.ShapeDtypeStruct((pad_n, c), jnp.float32),
        compiler_params=pltpu.CompilerParams(
            dimension_semantics=("arbitrary",)),
    )(texp, hm_g, w1, w3, w2)


def _combine_body(x2_ref, g0_ref, g1_ref, wt_ref, out_ref):
    w0 = wt_ref[...][:, :1]
    w1 = wt_ref[...][:, 1:2]
    out_ref[...] = x2_ref[...] + w0 * g0_ref[...] + w1 * g1_ref[...]


def _combine(x2, g0, g1, wt2, tm):
    n, c = x2.shape
    spec = pl.BlockSpec((tm, c), lambda i: (i, 0))
    return pl.pallas_call(
        _combine_body,
        grid=(n // tm,),
        in_specs=[spec, spec, spec,
                  pl.BlockSpec((tm, 128), lambda i: (i, 0))],
        out_specs=spec,
        out_shape=jax.ShapeDtypeStruct((n, c), jnp.float32),
        compiler_params=pltpu.CompilerParams(
            dimension_semantics=("parallel",)),
    )(x2, g0, g1, wt2)


from jax.experimental.pallas import tpu_sc as plsc


def _sc_dispatch(hm, d0, d1, pad_n):
    n, c = hm.shape
    nw, chunks, ch = d0.shape
    info = plsc.get_sparse_core_info()
    nc = info.num_cores
    mesh = plsc.VectorSubcoreMesh(core_axis_name="c", subcore_axis_name="s")

    @functools.partial(
        pl.kernel, mesh=mesh,
        out_type=jax.ShapeDtypeStruct((pad_n, c), jnp.float32),
        scratch_types=[
            pltpu.VMEM((chunks, ch), jnp.int32),
            pltpu.VMEM((chunks, ch), jnp.int32),
            pltpu.VMEM((ch, c), jnp.float32),
            pltpu.SemaphoreType.DMA,
        ],
    )
    def k(hm_hbm, d0_hbm, d1_hbm, out_hbm, i0_v, i1_v, rows_v, sem):
        wid = jax.lax.axis_index("s") * nc + jax.lax.axis_index("c")
        pltpu.sync_copy(d0_hbm.at[wid], i0_v)
        pltpu.sync_copy(d1_hbm.at[wid], i1_v)
        rows_per_w = chunks * ch

        def body(ci, carry):
            base = wid * rows_per_w + ci * ch
            pltpu.sync_copy(hm_hbm.at[pl.ds(base, ch)], rows_v)
            pltpu.async_copy(rows_v, out_hbm.at[i0_v.at[ci]], sem).wait()
            pltpu.async_copy(rows_v, out_hbm.at[i1_v.at[ci]], sem).wait()
            return carry

        jax.lax.fori_loop(0, chunks, body, 0)

    return k(hm, d0, d1)


def _sc_gather(table, idx3):
    nw, chunks, ch = idx3.shape
    v, d = table.shape
    bsz = nw * chunks * ch
    info = plsc.get_sparse_core_info()
    nc = info.num_cores
    mesh = plsc.VectorSubcoreMesh(core_axis_name="c", subcore_axis_name="s")

    @functools.partial(
        pl.kernel, mesh=mesh,
        out_type=jax.ShapeDtypeStruct((bsz, d), jnp.float32),
        scratch_types=[
            pltpu.VMEM((chunks, ch), jnp.int32),
            pltpu.VMEM((ch, d), jnp.float32),
            pltpu.SemaphoreType.DMA,
        ],
    )
    def k(table_hbm, idx_hbm, out_hbm, idx_v, rows_v, sem):
        wid = jax.lax.axis_index("s") * nc + jax.lax.axis_index("c")
        pltpu.sync_copy(idx_hbm.at[wid], idx_v)
        base = wid * (chunks * ch)

        def body(ci, carry):
            pltpu.async_copy(table_hbm.at[idx_v.at[ci]], rows_v, sem).wait()
            pltpu.sync_copy(rows_v, out_hbm.at[pl.ds(base + ci * ch, ch)])
            return carry

        jax.lax.fori_loop(0, chunks, body, 0)

    return k(table, idx3)


def kernel(x, rope_cos, rope_sin, attn_norm_w, q_w, k_w, v_w, o_w,
           ffn_norm_w, router_w, w1, w2, w3):
    b, t, c = x.shape
    dh = rope_cos.shape[1]
    h = c // dh
    n = b * t
    ne = router_w.shape[1]

    xf = x.reshape(n, c)
    tm = min(512, t)
    qh, kh, vx = _qkv_rope(x, attn_norm_w, q_w, k_w, v_w,
                           rope_cos, rope_sin, tm, h)
    tq = min(2048, t)
    tk = min(1024, t)
    y = _flash(qh.reshape(b * h, t, dh), kh.reshape(b * h, t, dh),
               vx.reshape(b * h, t, 2 * dh), tq, tk)
    yf = y

    x2, hm, idx2, wt2 = _post(xf, yf.reshape(b, h, t, dh), o_w, ffn_norm_w,
                              router_w, tm)

    tg = 512
    pad_n = 2 * n + ne * tg
    nw = 32
    i0 = idx2[:, 0].reshape(n // 128, 128)
    i1 = idx2[:, 1].reshape(n // 128, 128)
    d0, d1, tx = _dispatch_meta(i0, i1, ne, tg)
    texp = tx[0, : pad_n // tg]
    rpw = n // nw
    ch = min(32, rpw)
    hm_g = _sc_dispatch(hm, d0.reshape(nw, rpw // ch, ch),
                        d1.reshape(nw, rpw // ch, ch), pad_n)
    rows = _grouped_ffn(hm_g, texp, w1, w2, w3, tg)
    p01 = jnp.concatenate([d0.reshape(n), d1.reshape(n)])
    ch2 = min(64, (2 * n) // nw)
    g = _sc_gather(rows, p01.reshape(nw, (2 * n) // (nw * ch2), ch2))
    out = _combine(x2, g[:n], g[n:], wt2, tm)
    return out.reshape(b, t, c)


# speedup vs baseline: 1.0275x; 1.0087x over previous
import functools
import jax
import jax.numpy as jnp
from jax.experimental import pallas as pl
from jax.experimental.pallas import tpu as pltpu

_NEG = -1e30


def _qkv_body(x_ref, nw_ref, qw_ref, kw_ref, vw_ref, q_ref, k_ref, v_ref):
    xb = x_ref[...]
    var = jnp.mean(xb * xb, axis=1, keepdims=True)
    h = (nw_ref[...] * xb) * jax.lax.rsqrt(var + 1e-6)
    q_ref[...] = jnp.dot(h, qw_ref[...], preferred_element_type=jnp.float32)
    k_ref[...] = jnp.dot(h, kw_ref[...], preferred_element_type=jnp.float32)
    v_ref[...] = jnp.dot(h, vw_ref[...], preferred_element_type=jnp.float32)


def _qkv(xf, attn_norm_w, q_w, k_w, v_w, tm):
    n, c = xf.shape
    grid = (n // tm,)
    wspec = pl.BlockSpec((c, c), lambda i: (0, 0))
    return pl.pallas_call(
        _qkv_body,
        grid=grid,
        in_specs=[
            pl.BlockSpec((tm, c), lambda i: (i, 0)),
            pl.BlockSpec((1, c), lambda i: (0, 0)),
            wspec, wspec, wspec,
        ],
        out_specs=[pl.BlockSpec((tm, c), lambda i: (i, 0))] * 3,
        out_shape=[jax.ShapeDtypeStruct((n, c), jnp.float32)] * 3,
        compiler_params=pltpu.CompilerParams(
            dimension_semantics=("parallel",)),
    )(xf, attn_norm_w.reshape(1, c), q_w, k_w, v_w)


def _qkvr_body(x_ref, nw_ref, qw_ref, kw_ref, vw_ref, c_ref, s_ref,
               qo_ref, ko_ref, vx_ref):
    xb = x_ref[0]
    var = jnp.mean(xb * xb, axis=1, keepdims=True)
    hh = (nw_ref[...] * xb) * jax.lax.rsqrt(var + 1e-6)
    q = jnp.dot(hh, qw_ref[...], preferred_element_type=jnp.float32)
    k = jnp.dot(hh, kw_ref[...], preferred_element_type=jnp.float32)
    v = jnp.dot(hh, vw_ref[...], preferred_element_type=jnp.float32)
    tm, c = q.shape
    hd = c_ref.shape[-1]
    nh = c // hd
    cos = c_ref[...][:, None, :]
    sin = s_ref[...][:, None, :]

    def rot(x2):
        x3 = x2.reshape(tm, nh, hd)
        x1 = x3[..., : hd // 2]
        xr = x3[..., hd // 2:]
        r = jnp.concatenate([-xr, x1], axis=-1)
        return (x3 * cos + r * sin).transpose(1, 0, 2)

    qo_ref[0] = rot(q)
    ko_ref[0] = rot(k)
    vt = v.reshape(tm, nh, hd).transpose(1, 0, 2)
    lane = jax.lax.broadcasted_iota(jnp.int32, (nh, tm, 2 * hd), 2)
    vx_ref[0] = jnp.where(lane < hd,
                          jnp.pad(vt, ((0, 0), (0, 0), (0, hd))),
                          jnp.where(lane == hd, 1.0, 0.0))


def _qkv_rope(x3, attn_norm_w, q_w, k_w, v_w, cos, sin, tm, nh):
    b, t, c = x3.shape
    dh = c // nh
    wspec = pl.BlockSpec((c, c), lambda bb, i: (0, 0))
    cspec = pl.BlockSpec((tm, dh), lambda bb, i: (i, 0))
    ospec = pl.BlockSpec((1, nh, tm, dh), lambda bb, i: (bb, 0, i, 0))
    vxspec = pl.BlockSpec((1, nh, tm, 2 * dh), lambda bb, i: (bb, 0, i, 0))
    return pl.pallas_call(
        _qkvr_body,
        grid=(b, t // tm),
        in_specs=[
            pl.BlockSpec((1, tm, c), lambda bb, i: (bb, i, 0)),
            pl.BlockSpec((1, c), lambda bb, i: (0, 0)),
            wspec, wspec, wspec, cspec, cspec,
        ],
        out_specs=[ospec, ospec, vxspec],
        out_shape=[jax.ShapeDtypeStruct((b, nh, t, dh), jnp.float32)] * 2
        + [jax.ShapeDtypeStruct((b, nh, t, 2 * dh), jnp.float32)],
        compiler_params=pltpu.CompilerParams(
            dimension_semantics=("parallel", "parallel")),
    )(x3, attn_norm_w.reshape(1, c), q_w, k_w, v_w, cos, sin)


def _flash_body(q_ref, k_ref, vx_ref, o_ref, acc_ref, *, scale, tq, tk):
    i = pl.program_id(1)
    j = pl.program_id(2)

    @pl.when(j == 0)
    def _init():
        acc_ref[...] = jnp.zeros_like(acc_ref)

    @pl.when(j * tk <= i * tq + tq - 1)
    def _compute():
        s = jax.lax.dot_general(q_ref[0], k_ref[0], (((1,), (1,)), ((), ())),
                                preferred_element_type=jnp.float32) * scale
        rows = jax.lax.broadcasted_iota(jnp.int32, (tq, tk), 0) + i * tq
        cols = jax.lax.broadcasted_iota(jnp.int32, (tq, tk), 1) + j * tk
        p = jnp.exp(jnp.where(rows >= cols, s, _NEG))
        acc_ref[...] += jnp.dot(p, vx_ref[0],
                                preferred_element_type=jnp.float32)

    @pl.when(j == pl.num_programs(2) - 1)
    def _finish():
        dh = o_ref.shape[-1]
        acc = acc_ref[...]
        o_ref[0] = acc[:, :dh] / acc[:, dh:dh + 1]


def _flash(q, k, vx, tq, tk):
    bh, t, dh = q.shape
    scale = 1.0 / (dh ** 0.5)
    grid = (bh, t // tq, t // tk)
    body = functools.partial(_flash_body, scale=scale, tq=tq, tk=tk)
    return pl.pallas_call(
        body,
        grid=grid,
        in_specs=[
            pl.BlockSpec((1, tq, dh), lambda w, i, j: (w, i, 0)),
            pl.BlockSpec((1, tk, dh), lambda w, i, j: (w, j, 0)),
            pl.BlockSpec((1, tk, 2 * dh), lambda w, i, j: (w, j, 0)),
        ],
        out_specs=pl.BlockSpec((1, tq, dh), lambda w, i, j: (w, i, 0)),
        out_shape=jax.ShapeDtypeStruct((bh, t, dh), jnp.float32),
        scratch_shapes=[
            pltpu.VMEM((tq, 2 * dh), jnp.float32),
        ],
        compiler_params=pltpu.CompilerParams(
            dimension_semantics=("parallel", "parallel", "arbitrary")),
    )(q, k, vx)


def _post_body(x_ref, y_ref, ow_ref, fw_ref, rw_ref,
               x2_ref, hm_ref, idx_ref, wt_ref, *, e):
    xb = x_ref[...]
    tm = xb.shape[0]
    yb = y_ref[0].transpose(1, 0, 2).reshape(tm, -1)
    x2 = xb + jnp.dot(yb, ow_ref[...],
                      preferred_element_type=jnp.float32)
    var = jnp.mean(x2 * x2, axis=1, keepdims=True)
    hm = (fw_ref[...] * x2) * jax.lax.rsqrt(var + 1e-6)
    logits = jnp.dot(hm, rw_ref[...], preferred_element_type=jnp.float32)
    tm, lanes = logits.shape
    colid = jax.lax.broadcasted_iota(jnp.int32, (tm, lanes), 1)
    lg = jnp.where(colid < e, logits, _NEG)
    m1 = jnp.max(lg, axis=1, keepdims=True)
    i1 = jnp.min(jnp.where(lg == m1, colid, 999), axis=1, keepdims=True)
    lg2 = jnp.where(colid == i1, _NEG, lg)
    m2 = jnp.max(lg2, axis=1, keepdims=True)
    i2 = jnp.min(jnp.where(lg2 == m2, colid, 999), axis=1, keepdims=True)
    e2 = jnp.exp(m2 - m1)
    w1v = 1.0 / (1.0 + e2)
    w2v = e2 / (1.0 + e2)
    x2_ref[...] = x2
    hm_ref[...] = hm
    idx_ref[...] = jnp.where(colid == 0, i1, jnp.where(colid == 1, i2, 0))
    wt_ref[...] = jnp.where(colid == 0, w1v, jnp.where(colid == 1, w2v, 0.0))


def _post(xf, yf, o_w, ffn_norm_w, router_w, tm):
    n, c = xf.shape
    b, nh, t, dh = yf.shape
    e = router_w.shape[1]
    rw = jnp.pad(router_w, ((0, 0), (0, 128 - e)))
    grid = (n // tm,)
    tpb = t // tm
    body = functools.partial(_post_body, e=e)
    return pl.pallas_call(
        body,
        grid=grid,
        in_specs=[
            pl.BlockSpec((tm, c), lambda i: (i, 0)),
            pl.BlockSpec((1, nh, tm, dh), lambda i: (i // tpb, 0, i % tpb, 0)),
            pl.BlockSpec((c, c), lambda i: (0, 0)),
            pl.BlockSpec((1, c), lambda i: (0, 0)),
            pl.BlockSpec((c, 128), lambda i: (0, 0)),
        ],
        out_specs=[
            pl.BlockSpec((tm, c), lambda i: (i, 0)),
            pl.BlockSpec((tm, c), lambda i: (i, 0)),
            pl.BlockSpec((tm, 128), lambda i: (i, 0)),
            pl.BlockSpec((tm, 128), lambda i: (i, 0)),
        ],
        out_shape=[
            jax.ShapeDtypeStruct((n, c), jnp.float32),
            jax.ShapeDtypeStruct((n, c), jnp.float32),
            jax.ShapeDtypeStruct((n, 128), jnp.int32),
            jax.ShapeDtypeStruct((n, 128), jnp.float32),
        ],
        compiler_params=pltpu.CompilerParams(
            dimension_semantics=("parallel",)),
    )(xf, yf, o_w, ffn_norm_w.reshape(1, c), rw)


def _meta_body(i0_ref, i1_ref, d0_ref, d1_ref, tx_ref, *, ne, tm):
    i0 = i0_ref[...]
    i1 = i1_ref[...]
    r, lanes = i0.shape
    ci = jax.lax.broadcasted_iota(jnp.int32, (lanes, lanes), 0)
    cj = jax.lax.broadcasted_iota(jnp.int32, (lanes, lanes), 1)
    ustrict = jnp.where(ci < cj, 1.0, 0.0)
    ri = jax.lax.broadcasted_iota(jnp.int32, (r, r), 0)
    rj = jax.lax.broadcasted_iota(jnp.int32, (r, r), 1)
    lstrict = jnp.where(rj < ri, 1.0, 0.0)
    d0 = jnp.zeros((r, lanes), jnp.float32)
    d1 = jnp.zeros((r, lanes), jnp.float32)
    tilei = (jax.lax.broadcasted_iota(jnp.int32, (1, 128), 1)
             .astype(jnp.float32) * tm)
    tx = jnp.zeros((1, 128), jnp.float32)
    poff = 0.0
    for e in range(ne):
        m0 = jnp.where(i0 == e, 1.0, 0.0)
        m1 = jnp.where(i1 == e, 1.0, 0.0)
        mm = m0 + m1
        colex = jnp.dot(mm, ustrict, preferred_element_type=jnp.float32)
        rowsum = jnp.sum(mm, axis=1, keepdims=True)
        rowex = jnp.dot(lstrict, rowsum, preferred_element_type=jnp.float32)
        cnt_before = colex + rowex
        count = jnp.sum(mm)
        d0 = jnp.where(i0 == e, poff + cnt_before, d0)
        d1 = jnp.where(i1 == e, poff + cnt_before + m0, d1)
        poff = poff + jnp.ceil(count / tm) * tm
        tx = tx + jnp.where(tilei >= poff, 1.0, 0.0)
    d0_ref[...] = d0.astype(jnp.int32)
    d1_ref[...] = d1.astype(jnp.int32)
    tx_ref[...] = jnp.minimum(tx, ne - 1).astype(jnp.int32)


def _dispatch_meta(i0, i1, ne, tm):
    r, lanes = i0.shape
    body = functools.partial(_meta_body, ne=ne, tm=tm)
    return pl.pallas_call(
        body,
        out_shape=[
            jax.ShapeDtypeStruct((r, lanes), jnp.int32),
            jax.ShapeDtypeStruct((r, lanes), jnp.int32),
            jax.ShapeDtypeStruct((1, 128), jnp.int32),
        ],
    )(i0, i1)


def _gffn_body(texp_ref, hm_ref, w1_ref, w3_ref, w2_ref, out_ref):
    hmb = hm_ref[...]
    a = jnp.dot(hmb, w1_ref[0], preferred_element_type=jnp.float32)
    b3 = jnp.dot(hmb, w3_ref[0], preferred_element_type=jnp.float32)
    act = (a * (1.0 / (1.0 + jnp.exp(-a)))) * b3
    out_ref[...] = jnp.dot(act, w2_ref[0], preferred_element_type=jnp.float32)


def _grouped_ffn(hm_g, texp, w1, w2, w3, tm):
    pad_n, c = hm_g.shape
    ne, _, hid = w1.shape
    mt = pad_n // tm
    grid_spec = pltpu.PrefetchScalarGridSpec(
        num_scalar_prefetch=1,
        grid=(mt,),
        in_specs=[
            pl.BlockSpec((tm, c), lambda i, texp: (i, 0)),
            pl.BlockSpec((1, c, hid), lambda i, texp: (texp[i], 0, 0)),
            pl.BlockSpec((1, c, hid), lambda i, texp: (texp[i], 0, 0)),
            pl.BlockSpec((1, hid, c), lambda i, texp: (texp[i], 0, 0)),
        ],
        out_specs=pl.BlockSpec((tm, c), lambda i, texp: (i, 0)),
    )
    return pl.pallas_call(
        _gffn_body,
        grid_spec=grid_spec,
        out_shape=jax.ShapeDtypeStruct((pad_n, c), jnp.float32),
        compiler_params=pltpu.CompilerParams(
            dimension_semantics=("arbitrary",)),
    )(texp, hm_g, w1, w3, w2)


def _combine_body(x2_ref, g0_ref, g1_ref, wt_ref, out_ref):
    w0 = wt_ref[...][:, :1]
    w1 = wt_ref[...][:, 1:2]
    out_ref[...] = x2_ref[...] + w0 * g0_ref[...] + w1 * g1_ref[...]


def _combine(x2, g0, g1, wt2, tm):
    n, c = x2.shape
    spec = pl.BlockSpec((tm, c), lambda i: (i, 0))
    return pl.pallas_call(
        _combine_body,
        grid=(n // tm,),
        in_specs=[spec, spec, spec,
                  pl.BlockSpec((tm, 128), lambda i: (i, 0))],
        out_specs=spec,
        out_shape=jax.ShapeDtypeStruct((n, c), jnp.float32),
        compiler_params=pltpu.CompilerParams(
            dimension_semantics=("parallel",)),
    )(x2, g0, g1, wt2)


from jax.experimental.pallas import tpu_sc as plsc


def _sc_dispatch(hm, d0, d1, pad_n):
    n, c = hm.shape
    nw, chunks, ch = d0.shape
    info = plsc.get_sparse_core_info()
    nc = info.num_cores
    mesh = plsc.VectorSubcoreMesh(core_axis_name="c", subcore_axis_name="s")

    @functools.partial(
        pl.kernel, mesh=mesh,
        out_type=jax.ShapeDtypeStruct((pad_n, c), jnp.float32),
        scratch_types=[
            pltpu.VMEM((chunks, ch), jnp.int32),
            pltpu.VMEM((chunks, ch), jnp.int32),
            pltpu.VMEM((ch, c), jnp.float32),
            pltpu.SemaphoreType.DMA,
        ],
    )
    def k(hm_hbm, d0_hbm, d1_hbm, out_hbm, i0_v, i1_v, rows_v, sem):
        wid = jax.lax.axis_index("s") * nc + jax.lax.axis_index("c")
        pltpu.sync_copy(d0_hbm.at[wid], i0_v)
        pltpu.sync_copy(d1_hbm.at[wid], i1_v)
        rows_per_w = chunks * ch

        def body(ci, carry):
            base = wid * rows_per_w + ci * ch
            pltpu.sync_copy(hm_hbm.at[pl.ds(base, ch)], rows_v)
            pltpu.async_copy(rows_v, out_hbm.at[i0_v.at[ci]], sem).wait()
            pltpu.async_copy(rows_v, out_hbm.at[i1_v.at[ci]], sem).wait()
            return carry

        jax.lax.fori_loop(0, chunks, body, 0)

    return k(hm, d0, d1)


def _sc_gather(table, idx3):
    nw, chunks, ch = idx3.shape
    v, d = table.shape
    bsz = nw * chunks * ch
    info = plsc.get_sparse_core_info()
    nc = info.num_cores
    mesh = plsc.VectorSubcoreMesh(core_axis_name="c", subcore_axis_name="s")

    @functools.partial(
        pl.kernel, mesh=mesh,
        out_type=jax.ShapeDtypeStruct((bsz, d), jnp.float32),
        scratch_types=[
            pltpu.VMEM((chunks, ch), jnp.int32),
            pltpu.VMEM((ch, d), jnp.float32),
            pltpu.SemaphoreType.DMA,
        ],
    )
    def k(table_hbm, idx_hbm, out_hbm, idx_v, rows_v, sem):
        wid = jax.lax.axis_index("s") * nc + jax.lax.axis_index("c")
        pltpu.sync_copy(idx_hbm.at[wid], idx_v)
        base = wid * (chunks * ch)

        def body(ci, carry):
            pltpu.async_copy(table_hbm.at[idx_v.at[ci]], rows_v, sem).wait()
            pltpu.sync_copy(rows_v, out_hbm.at[pl.ds(base + ci * ch, ch)])
            return carry

        jax.lax.fori_loop(0, chunks, body, 0)

    return k(table, idx3)


def kernel(x, rope_cos, rope_sin, attn_norm_w, q_w, k_w, v_w, o_w,
           ffn_norm_w, router_w, w1, w2, w3):
    b, t, c = x.shape
    dh = rope_cos.shape[1]
    h = c // dh
    n = b * t
    ne = router_w.shape[1]

    xf = x.reshape(n, c)
    tm = min(512, t)
    qh, kh, vx = _qkv_rope(x, attn_norm_w, q_w, k_w, v_w,
                           rope_cos, rope_sin, tm, h)
    tq = min(2048, t)
    tk = min(1024, t)
    y = _flash(qh.reshape(b * h, t, dh), kh.reshape(b * h, t, dh),
               vx.reshape(b * h, t, 2 * dh), tq, tk)
    yf = y

    x2, hm, idx2, wt2 = _post(xf, yf.reshape(b, h, t, dh), o_w, ffn_norm_w,
                              router_w, tm)

    tg = 256
    pad_n = 2 * n + ne * tg
    nw = 32
    i0 = idx2[:, 0].reshape(n // 128, 128)
    i1 = idx2[:, 1].reshape(n // 128, 128)
    d0, d1, tx = _dispatch_meta(i0, i1, ne, tg)
    texp = tx[0, : pad_n // tg]
    rpw = n // nw
    ch = min(32, rpw)
    hm_g = _sc_dispatch(hm, d0.reshape(nw, rpw // ch, ch),
                        d1.reshape(nw, rpw // ch, ch), pad_n)
    rows = _grouped_ffn(hm_g, texp, w1, w2, w3, tg)
    p01 = jnp.concatenate([d0.reshape(n), d1.reshape(n)])
    ch2 = min(64, (2 * n) // nw)
    g = _sc_gather(rows, p01.reshape(nw, (2 * n) // (nw * ch2), ch2))
    out = _combine(x2, g[:n], g[n:], wt2, tm)
    return out.reshape(b, t, c)
